# Initial kernel scaffold; baseline (speedup 1.0000x reference)
import functools

import jax
import jax.numpy as jnp
from jax import lax
from jax.experimental import pallas as pl
from jax.experimental.pallas import tpu as pltpu
from jax.experimental.pallas import tpu_sc as plsc

N = 10000
E = 320000
DIN = 128
DH = 64
DO = 32
G = 16

NC = 2
NS = 16
NW = NC * NS
CH = 125
K = 8
EPW = E // NW
T = EPW // (K * CH)
RPT = N // NS

_mesh = plsc.VectorSubcoreMesh(core_axis_name="c", subcore_axis_name="s")


def _make_prop(d):
    @functools.partial(
        pl.kernel,
        out_type=jax.ShapeDtypeStruct((NC, N, d), jnp.float32),
        mesh=_mesh,
        scratch_types=[
            pltpu.VMEM((K, CH), jnp.int32),
            pltpu.VMEM((K, CH), jnp.int32),
            pltpu.VMEM((K, CH, d), jnp.float32),
            pltpu.MemorySpace.VMEM_SHARED((N, d), jnp.float32),
            pltpu.SemaphoreType.DMA,
            pltpu.SemaphoreType.DMA,
        ],
    )
    def prop(src_hbm, dst_hbm, xs_hbm, zeros_hbm, out_hbm,
             idx_s, idx_d, rows, acc, sem_g, sem_s):
        c = lax.axis_index("c")
        s = lax.axis_index("s")
        wid = c * NS + s
        pltpu.sync_copy(zeros_hbm, acc.at[pl.ds(s * RPT, RPT)])
        plsc.subcore_barrier()

        row0 = wid * (EPW // CH)

        def step(t, carry):
            r = row0 + t * K
            pltpu.sync_copy(src_hbm.at[pl.ds(r, K)], idx_s)
            pltpu.sync_copy(dst_hbm.at[pl.ds(r, K)], idx_d)
            gets = [pltpu.async_copy(xs_hbm.at[idx_s.at[j]], rows.at[j], sem_g)
                    for j in range(K)]
            for g in gets:
                g.wait()
            puts = [pltpu.async_copy(rows.at[j], acc.at[idx_d.at[j]], sem_s,
                                     add=True)
                    for j in range(K)]
            for p in puts:
                p.wait()
            return carry

        lax.fori_loop(0, T, step, 0)
        plsc.subcore_barrier()
        pltpu.sync_copy(acc.at[pl.ds(s * RPT, RPT)],
                        out_hbm.at[c, pl.ds(s * RPT, RPT)])

    return prop


_prop64 = _make_prop(DH)
_prop32 = _make_prop(DO)


@functools.partial(
    pl.kernel,
    out_type=jax.ShapeDtypeStruct((NC, N, 16), jnp.float32),
    mesh=_mesh,
    scratch_types=[
        pltpu.VMEM((K, CH), jnp.int32),
        pltpu.VMEM((CH, 16), jnp.float32),
        pltpu.MemorySpace.VMEM_SHARED((N, 16), jnp.float32),
        pltpu.SemaphoreType.DMA,
    ],
)
def _deg(dst_hbm, ones_hbm, zeros_hbm, out_hbm, idx_d, ones_v, acc, sem_s):
    c = lax.axis_index("c")
    s = lax.axis_index("s")
    wid = c * NS + s
    pltpu.sync_copy(ones_hbm, ones_v)
    pltpu.sync_copy(zeros_hbm, acc.at[pl.ds(s * RPT, RPT)])
    plsc.subcore_barrier()

    row0 = wid * (EPW // CH)

    def step(t, carry):
        r = row0 + t * K
        pltpu.sync_copy(dst_hbm.at[pl.ds(r, K)], idx_d)
        puts = [pltpu.async_copy(ones_v, acc.at[idx_d.at[j]], sem_s, add=True)
                for j in range(K)]
        for p in puts:
            p.wait()
        return carry

    lax.fori_loop(0, T, step, 0)
    plsc.subcore_barrier()
    pltpu.sync_copy(acc.at[pl.ds(s * RPT, RPT)],
                    out_hbm.at[c, pl.ds(s * RPT, RPT)])


def _tc1_body(x_ref, w1_ref, degp_ref, xs1_ref, dinvb_ref):
    deg = degp_ref[0, :, 0:1] + degp_ref[1, :, 0:1] + 1.0
    dinv = lax.rsqrt(deg)
    xw = jnp.dot(x_ref[...], w1_ref[...], preferred_element_type=jnp.float32)
    xs1_ref[...] = xw * dinv
    dinvb_ref[...] = jnp.broadcast_to(dinv, (N, DH))


def _tc2_body(acc_ref, xs1_ref, dinvb_ref, b1_ref, w2_ref, xs2_ref):
    tot = acc_ref[0] + acc_ref[1] + xs1_ref[...]
    h1 = jnp.maximum(dinvb_ref[...] * tot + b1_ref[...], 0.0)
    xs2_ref[...] = (jnp.dot(h1, w2_ref[...], preferred_element_type=jnp.float32)
                    * dinvb_ref[:, :DO])


def _tc3_body(acc_ref, xs2_ref, dinvb_ref, b2_ref, batch_ref, scal_ref,
              w3_ref, b3_ref, w4_ref, b4_ref, out_ref):
    tot = acc_ref[0] + acc_ref[1] + xs2_ref[...]
    h2 = dinvb_ref[:, :DO] * tot + b2_ref[...]
    gids = lax.broadcasted_iota(jnp.int32, (N, G), 1)
    p = (batch_ref[...] == gids).astype(jnp.float32)
    sums = lax.dot_general(p, h2, (((0,), (0,)), ((), ())),
                           preferred_element_type=jnp.float32)
    counts = lax.dot_general(p, jnp.ones((N, 1), jnp.float32),
                             (((0,), (0,)), ((), ())),
                             preferred_element_type=jnp.float32)
    pooled = sums / jnp.maximum(counts, 1.0)
    comb = jnp.concatenate([pooled, scal_ref[...]], axis=1)
    z = jnp.maximum(
        jnp.dot(comb, w3_ref[...], preferred_element_type=jnp.float32)
        + b3_ref[...], 0.0)
    o = jnp.dot(z, w4_ref[...], preferred_element_type=jnp.float32) + b4_ref[...]
    out_ref[...] = 1.0 / (1.0 + jnp.exp(-o))


_tc1 = pl.pallas_call(
    _tc1_body,
    out_shape=[jax.ShapeDtypeStruct((N, DH), jnp.float32),
               jax.ShapeDtypeStruct((N, DH), jnp.float32)],
)

_tc2 = pl.pallas_call(
    _tc2_body,
    out_shape=jax.ShapeDtypeStruct((N, DO), jnp.float32),
)

_tc3 = pl.pallas_call(
    _tc3_body,
    out_shape=jax.ShapeDtypeStruct((G, 1), jnp.float32),
)


def kernel(x, edge_index, batch, scalars, W1, b1, W2, b2, W3, b3, W4, b4):
    src2d = edge_index[0].astype(jnp.int32).reshape(E // CH, CH)
    dst2d = edge_index[1].astype(jnp.int32).reshape(E // CH, CH)
    ones16 = jnp.ones((CH, 16), jnp.float32)
    z16 = jnp.zeros((RPT, 16), jnp.float32)
    z64 = jnp.zeros((RPT, DH), jnp.float32)
    z32 = jnp.zeros((RPT, DO), jnp.float32)

    degp = _deg(dst2d, ones16, z16)
    xs1, dinvb = _tc1(x, W1, degp)
    acc1 = _prop64(src2d, dst2d, xs1, z64)
    xs2 = _tc2(acc1, xs1, dinvb, b1.reshape(1, DH), W2)
    acc2 = _prop32(src2d, dst2d, xs2, z32)
    out = _tc3(acc2, xs2, dinvb, b2.reshape(1, DO),
               batch.astype(jnp.int32).reshape(N, 1), scalars,
               W3, b3.reshape(1, DH), W4, b4.reshape(1, 1))
    return out


# speedup vs baseline: 36.9587x; 36.9587x over previous
import functools

import jax
import jax.numpy as jnp
from jax import lax
from jax.experimental import pallas as pl
from jax.experimental.pallas import tpu as pltpu
from jax.experimental.pallas import tpu_sc as plsc

N = 10000
E = 320000
DIN = 128
DH = 64
DO = 32
G = 16

NC = 2
NS = 16
NW = NC * NS
CH = 125
K = 8
EPW = E // NW
T = EPW // (K * CH)
RPT = 624
REM = N - NS * RPT

_mesh = plsc.VectorSubcoreMesh(core_axis_name="c", subcore_axis_name="s")
_sc_params = pltpu.CompilerParams(use_tc_tiling_on_sc=False)


def _make_prop(d):
    @functools.partial(
        pl.kernel,
        out_type=jax.ShapeDtypeStruct((NC, N, d), jnp.float32),
        mesh=_mesh,
        compiler_params=_sc_params,
        scratch_types=[
            pltpu.VMEM((K, CH), jnp.int32),
            pltpu.VMEM((K, CH), jnp.int32),
            pltpu.VMEM((K, CH, d), jnp.float32),
            pltpu.MemorySpace.VMEM_SHARED((N, d), jnp.float32),
            pltpu.SemaphoreType.DMA,
            pltpu.SemaphoreType.DMA,
        ],
    )
    def prop(src_hbm, dst_hbm, xs_hbm, zeros_hbm, out_hbm,
             idx_s, idx_d, rows, acc, sem_g, sem_s):
        c = lax.axis_index("c")
        s = lax.axis_index("s")
        wid = c * NS + s
        pltpu.sync_copy(zeros_hbm, acc.at[pl.ds(s * RPT, RPT)])

        @pl.when(s == NS - 1)
        def _():
            pltpu.sync_copy(zeros_hbm.at[pl.ds(0, REM)],
                            acc.at[pl.ds(NS * RPT, REM)])

        plsc.subcore_barrier()

        row0 = wid * (EPW // CH)

        def step(t, carry):
            r = row0 + t * K
            pltpu.sync_copy(src_hbm.at[pl.ds(r, K)], idx_s)
            pltpu.sync_copy(dst_hbm.at[pl.ds(r, K)], idx_d)
            gets = [pltpu.async_copy(xs_hbm.at[idx_s.at[j]], rows.at[j], sem_g)
                    for j in range(K)]
            for g in gets:
                g.wait()
            puts = [pltpu.async_copy(rows.at[j], acc.at[idx_d.at[j]], sem_s,
                                     add=True)
                    for j in range(K)]
            for p in puts:
                p.wait()
            return carry

        lax.fori_loop(0, T, step, 0)
        plsc.subcore_barrier()
        pltpu.sync_copy(acc.at[pl.ds(s * RPT, RPT)],
                        out_hbm.at[c, pl.ds(s * RPT, RPT)])

        @pl.when(s == NS - 1)
        def _():
            pltpu.sync_copy(acc.at[pl.ds(NS * RPT, REM)],
                            out_hbm.at[c, pl.ds(NS * RPT, REM)])

    return prop


_prop64 = _make_prop(DH)
_prop32 = _make_prop(DO)


@functools.partial(
    pl.kernel,
    out_type=jax.ShapeDtypeStruct((NC, N, 16), jnp.float32),
    mesh=_mesh,
    compiler_params=_sc_params,
    scratch_types=[
        pltpu.VMEM((K, CH), jnp.int32),
        pltpu.VMEM((CH, 16), jnp.float32),
        pltpu.MemorySpace.VMEM_SHARED((N, 16), jnp.float32),
        pltpu.SemaphoreType.DMA,
    ],
)
def _deg(dst_hbm, ones_hbm, zeros_hbm, out_hbm, idx_d, ones_v, acc, sem_s):
    c = lax.axis_index("c")
    s = lax.axis_index("s")
    wid = c * NS + s
    pltpu.sync_copy(ones_hbm, ones_v)
    pltpu.sync_copy(zeros_hbm, acc.at[pl.ds(s * RPT, RPT)])

    @pl.when(s == NS - 1)
    def _():
        pltpu.sync_copy(zeros_hbm.at[pl.ds(0, REM)],
                        acc.at[pl.ds(NS * RPT, REM)])

    plsc.subcore_barrier()

    row0 = wid * (EPW // CH)

    def step(t, carry):
        r = row0 + t * K
        pltpu.sync_copy(dst_hbm.at[pl.ds(r, K)], idx_d)
        puts = [pltpu.async_copy(ones_v, acc.at[idx_d.at[j]], sem_s, add=True)
                for j in range(K)]
        for p in puts:
            p.wait()
        return carry

    lax.fori_loop(0, T, step, 0)
    plsc.subcore_barrier()
    pltpu.sync_copy(acc.at[pl.ds(s * RPT, RPT)],
                    out_hbm.at[c, pl.ds(s * RPT, RPT)])

    @pl.when(s == NS - 1)
    def _():
        pltpu.sync_copy(acc.at[pl.ds(NS * RPT, REM)],
                        out_hbm.at[c, pl.ds(NS * RPT, REM)])


def _tc1_body(x_ref, w1_ref, degp_ref, xs1_ref, dinvb_ref):
    deg = degp_ref[0, :, 0:1] + degp_ref[1, :, 0:1] + 1.0
    dinv = lax.rsqrt(deg)
    xw = jnp.dot(x_ref[...], w1_ref[...], preferred_element_type=jnp.float32)
    xs1_ref[...] = xw * dinv
    dinvb_ref[...] = jnp.broadcast_to(dinv, (N, DH))


def _tc2_body(acc_ref, xs1_ref, dinvb_ref, b1_ref, w2_ref, xs2_ref):
    tot = acc_ref[0] + acc_ref[1] + xs1_ref[...]
    h1 = jnp.maximum(dinvb_ref[...] * tot + b1_ref[...], 0.0)
    xs2_ref[...] = (jnp.dot(h1, w2_ref[...], preferred_element_type=jnp.float32)
                    * dinvb_ref[:, :DO])


def _tc3_body(acc_ref, xs2_ref, dinvb_ref, b2_ref, batch_ref, scal_ref,
              w3_ref, b3_ref, w4_ref, b4_ref, out_ref):
    tot = acc_ref[0] + acc_ref[1] + xs2_ref[...]
    h2 = dinvb_ref[:, :DO] * tot + b2_ref[...]
    gids = lax.broadcasted_iota(jnp.int32, (N, G), 1)
    p = (batch_ref[...] == gids).astype(jnp.float32)
    sums = lax.dot_general(p, h2, (((0,), (0,)), ((), ())),
                           preferred_element_type=jnp.float32)
    counts = lax.dot_general(p, jnp.ones((N, 1), jnp.float32),
                             (((0,), (0,)), ((), ())),
                             preferred_element_type=jnp.float32)
    pooled = sums / jnp.maximum(counts, 1.0)
    comb = jnp.concatenate([pooled, scal_ref[...]], axis=1)
    z = jnp.maximum(
        jnp.dot(comb, w3_ref[...], preferred_element_type=jnp.float32)
        + b3_ref[...], 0.0)
    o = jnp.dot(z, w4_ref[...], preferred_element_type=jnp.float32) + b4_ref[...]
    out_ref[...] = 1.0 / (1.0 + jnp.exp(-o))


_tc1 = pl.pallas_call(
    _tc1_body,
    out_shape=[jax.ShapeDtypeStruct((N, DH), jnp.float32),
               jax.ShapeDtypeStruct((N, DH), jnp.float32)],
)

_tc2 = pl.pallas_call(
    _tc2_body,
    out_shape=jax.ShapeDtypeStruct((N, DO), jnp.float32),
)

_tc3 = pl.pallas_call(
    _tc3_body,
    out_shape=jax.ShapeDtypeStruct((G, 1), jnp.float32),
)


def kernel(x, edge_index, batch, scalars, W1, b1, W2, b2, W3, b3, W4, b4):
    src2d = edge_index[0].astype(jnp.int32).reshape(E // CH, CH)
    dst2d = edge_index[1].astype(jnp.int32).reshape(E // CH, CH)
    ones16 = jnp.ones((CH, 16), jnp.float32)
    z16 = jnp.zeros((RPT, 16), jnp.float32)
    z64 = jnp.zeros((RPT, DH), jnp.float32)
    z32 = jnp.zeros((RPT, DO), jnp.float32)

    degp = _deg(dst2d, ones16, z16)
    xs1, dinvb = _tc1(x, W1, degp)
    acc1 = _prop64(src2d, dst2d, xs1, z64)
    xs2 = _tc2(acc1, xs1, dinvb, b1.reshape(1, DH), W2)
    acc2 = _prop32(src2d, dst2d, xs2, z32)
    out = _tc3(acc2, xs2, dinvb, b2.reshape(1, DO),
               batch.astype(jnp.int32).reshape(N, 1), scalars,
               W3, b3.reshape(1, DH), W4, b4.reshape(1, 1))
    return out


# speedup vs baseline: 46.4004x; 1.2555x over previous
import functools

import jax
import jax.numpy as jnp
from jax import lax
from jax.experimental import pallas as pl
from jax.experimental.pallas import tpu as pltpu
from jax.experimental.pallas import tpu_sc as plsc

N = 10000
E = 320000
DIN = 128
DH = 64
DO = 32
G = 16

NC = 2
NS = 16
NW = NC * NS
CH = 125
EPW = E // NW
EPR = EPW // CH
KP = 4
TP = EPR // KP
KD = 8
TD = EPR // KD
RPT = 624
REM = N - NS * RPT

_mesh = plsc.VectorSubcoreMesh(core_axis_name="c", subcore_axis_name="s")
_sc_params = pltpu.CompilerParams(use_tc_tiling_on_sc=False)


def _make_prop(d):
    @functools.partial(
        pl.kernel,
        out_type=jax.ShapeDtypeStruct((NC, N, d), jnp.float32),
        mesh=_mesh,
        compiler_params=_sc_params,
        scratch_types=[
            pltpu.VMEM((EPR, CH), jnp.int32),
            pltpu.VMEM((EPR, CH), jnp.int32),
            pltpu.VMEM((2, KP, CH, d), jnp.float32),
            pltpu.MemorySpace.VMEM_SHARED((N, d), jnp.float32),
            pltpu.SemaphoreType.DMA,
            pltpu.SemaphoreType.DMA,
        ],
    )
    def prop(src_hbm, dst_hbm, xs_hbm, zeros_hbm, out_hbm,
             idx_s, idx_d, rows, acc, sem_g, sem_s):
        c = lax.axis_index("c")
        s = lax.axis_index("s")
        wid = c * NS + s
        row0 = wid * EPR
        pltpu.sync_copy(src_hbm.at[pl.ds(row0, EPR)], idx_s)
        pltpu.sync_copy(dst_hbm.at[pl.ds(row0, EPR)], idx_d)
        pltpu.sync_copy(zeros_hbm, acc.at[pl.ds(s * RPT, RPT)])

        @pl.when(s == NS - 1)
        def _():
            pltpu.sync_copy(zeros_hbm.at[pl.ds(0, REM)],
                            acc.at[pl.ds(NS * RPT, REM)])

        plsc.subcore_barrier()

        def fire_gather(t, b):
            for j in range(KP):
                pltpu.async_copy(xs_hbm.at[idx_s.at[t * KP + j]],
                                 rows.at[b, j], sem_g)

        def drain_gather():
            for j in range(KP):
                pltpu.make_async_copy(xs_hbm.at[idx_s.at[0]],
                                      rows.at[0, j], sem_g).wait()

        def fire_scatter(t, b):
            for j in range(KP):
                pltpu.async_copy(rows.at[b, j], acc.at[idx_d.at[t * KP + j]],
                                 sem_s, add=True)

        def drain_scatter():
            for j in range(KP):
                pltpu.make_async_copy(rows.at[0, j],
                                      acc.at[idx_d.at[0]], sem_s).wait()

        fire_gather(0, 0)

        def step(t, carry):
            b = lax.rem(t, 2)
            nb = 1 - b

            @pl.when(t >= 1)
            def _():
                drain_scatter()

            @pl.when(t < TP - 1)
            def _():
                fire_gather(t + 1, nb)

            drain_gather()
            fire_scatter(t, b)
            return carry

        lax.fori_loop(0, TP, step, 0)
        drain_scatter()
        plsc.subcore_barrier()
        pltpu.sync_copy(acc.at[pl.ds(s * RPT, RPT)],
                        out_hbm.at[c, pl.ds(s * RPT, RPT)])

        @pl.when(s == NS - 1)
        def _():
            pltpu.sync_copy(acc.at[pl.ds(NS * RPT, REM)],
                            out_hbm.at[c, pl.ds(NS * RPT, REM)])

    return prop


_prop64 = _make_prop(DH)
_prop32 = _make_prop(DO)


@functools.partial(
    pl.kernel,
    out_type=jax.ShapeDtypeStruct((NC, N, 16), jnp.float32),
    mesh=_mesh,
    compiler_params=_sc_params,
    scratch_types=[
        pltpu.VMEM((EPR, CH), jnp.int32),
        pltpu.VMEM((CH, 16), jnp.float32),
        pltpu.MemorySpace.VMEM_SHARED((N, 16), jnp.float32),
        pltpu.SemaphoreType.DMA,
    ],
)
def _deg(dst_hbm, ones_hbm, zeros_hbm, out_hbm, idx_d, ones_v, acc, sem_s):
    c = lax.axis_index("c")
    s = lax.axis_index("s")
    wid = c * NS + s
    pltpu.sync_copy(ones_hbm, ones_v)
    pltpu.sync_copy(dst_hbm.at[pl.ds(wid * EPR, EPR)], idx_d)
    pltpu.sync_copy(zeros_hbm, acc.at[pl.ds(s * RPT, RPT)])

    @pl.when(s == NS - 1)
    def _():
        pltpu.sync_copy(zeros_hbm.at[pl.ds(0, REM)],
                        acc.at[pl.ds(NS * RPT, REM)])

    plsc.subcore_barrier()

    def drain():
        for j in range(KD):
            pltpu.make_async_copy(ones_v, acc.at[idx_d.at[0]], sem_s).wait()

    def step(t, carry):
        @pl.when(t >= 1)
        def _():
            drain()

        for j in range(KD):
            pltpu.async_copy(ones_v, acc.at[idx_d.at[t * KD + j]], sem_s,
                             add=True)
        return carry

    lax.fori_loop(0, TD, step, 0)
    drain()
    plsc.subcore_barrier()
    pltpu.sync_copy(acc.at[pl.ds(s * RPT, RPT)],
                    out_hbm.at[c, pl.ds(s * RPT, RPT)])

    @pl.when(s == NS - 1)
    def _():
        pltpu.sync_copy(acc.at[pl.ds(NS * RPT, REM)],
                        out_hbm.at[c, pl.ds(NS * RPT, REM)])


def _tc1_body(x_ref, w1_ref, degp_ref, xs1_ref, dinvb_ref):
    deg = degp_ref[0, :, 0:1] + degp_ref[1, :, 0:1] + 1.0
    dinv = lax.rsqrt(deg)
    xw = jnp.dot(x_ref[...], w1_ref[...], preferred_element_type=jnp.float32)
    xs1_ref[...] = xw * dinv
    dinvb_ref[...] = jnp.broadcast_to(dinv, (N, DH))


def _tc2_body(acc_ref, xs1_ref, dinvb_ref, b1_ref, w2_ref, xs2_ref):
    tot = acc_ref[0] + acc_ref[1] + xs1_ref[...]
    h1 = jnp.maximum(dinvb_ref[...] * tot + b1_ref[...], 0.0)
    xs2_ref[...] = (jnp.dot(h1, w2_ref[...], preferred_element_type=jnp.float32)
                    * dinvb_ref[:, :DO])


def _tc3_body(acc_ref, xs2_ref, dinvb_ref, b2_ref, batch_ref, scal_ref,
              w3_ref, b3_ref, w4_ref, b4_ref, out_ref):
    tot = acc_ref[0] + acc_ref[1] + xs2_ref[...]
    h2 = dinvb_ref[:, :DO] * tot + b2_ref[...]
    gids = lax.broadcasted_iota(jnp.int32, (N, G), 1)
    p = (batch_ref[...] == gids).astype(jnp.float32)
    sums = lax.dot_general(p, h2, (((0,), (0,)), ((), ())),
                           preferred_element_type=jnp.float32)
    counts = lax.dot_general(p, jnp.ones((N, 1), jnp.float32),
                             (((0,), (0,)), ((), ())),
                             preferred_element_type=jnp.float32)
    pooled = sums / jnp.maximum(counts, 1.0)
    comb = jnp.concatenate([pooled, scal_ref[...]], axis=1)
    z = jnp.maximum(
        jnp.dot(comb, w3_ref[...], preferred_element_type=jnp.float32)
        + b3_ref[...], 0.0)
    o = jnp.dot(z, w4_ref[...], preferred_element_type=jnp.float32) + b4_ref[...]
    out_ref[...] = 1.0 / (1.0 + jnp.exp(-o))


_tc1 = pl.pallas_call(
    _tc1_body,
    out_shape=[jax.ShapeDtypeStruct((N, DH), jnp.float32),
               jax.ShapeDtypeStruct((N, DH), jnp.float32)],
)

_tc2 = pl.pallas_call(
    _tc2_body,
    out_shape=jax.ShapeDtypeStruct((N, DO), jnp.float32),
)

_tc3 = pl.pallas_call(
    _tc3_body,
    out_shape=jax.ShapeDtypeStruct((G, 1), jnp.float32),
)


def kernel(x, edge_index, batch, scalars, W1, b1, W2, b2, W3, b3, W4, b4):
    src2d = edge_index[0].astype(jnp.int32).reshape(E // CH, CH)
    dst2d = edge_index[1].astype(jnp.int32).reshape(E // CH, CH)
    ones16 = jnp.ones((CH, 16), jnp.float32)
    z16 = jnp.zeros((RPT, 16), jnp.float32)
    z64 = jnp.zeros((RPT, DH), jnp.float32)
    z32 = jnp.zeros((RPT, DO), jnp.float32)

    degp = _deg(dst2d, ones16, z16)
    xs1, dinvb = _tc1(x, W1, degp)
    acc1 = _prop64(src2d, dst2d, xs1, z64)
    xs2 = _tc2(acc1, xs1, dinvb, b1.reshape(1, DH), W2)
    acc2 = _prop32(src2d, dst2d, xs2, z32)
    out = _tc3(acc2, xs2, dinvb, b2.reshape(1, DO),
               batch.astype(jnp.int32).reshape(N, 1), scalars,
               W3, b3.reshape(1, DH), W4, b4.reshape(1, 1))
    return out


# speedup vs baseline: 51.4213x; 1.1082x over previous
import functools

import jax
import jax.numpy as jnp
from jax import lax
from jax.experimental import pallas as pl
from jax.experimental.pallas import tpu as pltpu
from jax.experimental.pallas import tpu_sc as plsc

N = 10000
E = 320000
DIN = 128
DH = 64
DO = 32
G = 16

NC = 2
NS = 16
NW = NC * NS
CH = 125
EPW = E // NW
EPR = EPW // CH
KP = 4
TP = EPR // KP
KD = 8
TD = EPR // KD
RPT = 624
REM = N - NS * RPT

_mesh = plsc.VectorSubcoreMesh(core_axis_name="c", subcore_axis_name="s")
_sc_params = pltpu.CompilerParams(use_tc_tiling_on_sc=False)


@functools.partial(
    pl.kernel,
    out_type=jax.ShapeDtypeStruct((NC, N, DH), jnp.float32),
    mesh=_mesh,
    compiler_params=_sc_params,
    scratch_types=[
        pltpu.VMEM((EPR, CH), jnp.int32),
        pltpu.VMEM((EPR, CH), jnp.int32),
        pltpu.VMEM((2, KP, CH, DH), jnp.float32),
        pltpu.MemorySpace.VMEM_SHARED((N, DH), jnp.float32),
        pltpu.SemaphoreType.DMA,
        pltpu.SemaphoreType.DMA,
    ],
)
def _prop64(src_hbm, dst_hbm, xs_hbm, z64_hbm, out_hbm,
            idx_s, idx_d, rows, acc, sem_g, sem_s):
    c = lax.axis_index("c")
    s = lax.axis_index("s")
    wid = c * NS + s
    row0 = wid * EPR
    pltpu.sync_copy(src_hbm.at[pl.ds(row0, EPR)], idx_s)
    pltpu.sync_copy(dst_hbm.at[pl.ds(row0, EPR)], idx_d)
    pltpu.sync_copy(z64_hbm, acc.at[pl.ds(s * RPT, RPT)])

    @pl.when(s == NS - 1)
    def _():
        pltpu.sync_copy(z64_hbm.at[pl.ds(0, REM)],
                        acc.at[pl.ds(NS * RPT, REM)])

    plsc.subcore_barrier()

    def fire_gather(t, b):
        for j in range(KP):
            pltpu.async_copy(xs_hbm.at[idx_s.at[t * KP + j]],
                             rows.at[b, j], sem_g)

    def drain_gather():
        for j in range(KP):
            pltpu.make_async_copy(xs_hbm.at[idx_s.at[0]],
                                  rows.at[0, j], sem_g).wait()

    def fire_scatter(t, b):
        for j in range(KP):
            pltpu.async_copy(rows.at[b, j], acc.at[idx_d.at[t * KP + j]],
                             sem_s, add=True)

    def drain_scatter():
        for j in range(KP):
            pltpu.make_async_copy(rows.at[0, j],
                                  acc.at[idx_d.at[0]], sem_s).wait()

    fire_gather(0, 0)

    def step(t, carry):
        b = lax.rem(t, 2)
        nb = 1 - b

        @pl.when(t >= 1)
        def _():
            drain_scatter()

        @pl.when(t < TP - 1)
        def _():
            fire_gather(t + 1, nb)

        drain_gather()
        fire_scatter(t, b)
        return carry

    lax.fori_loop(0, TP, step, 0)
    drain_scatter()
    plsc.subcore_barrier()
    pltpu.sync_copy(acc.at[pl.ds(s * RPT, RPT)],
                    out_hbm.at[c, pl.ds(s * RPT, RPT)])

    @pl.when(s == NS - 1)
    def _():
        pltpu.sync_copy(acc.at[pl.ds(NS * RPT, REM)],
                        out_hbm.at[c, pl.ds(NS * RPT, REM)])


@functools.partial(
    pl.kernel,
    out_type=jax.ShapeDtypeStruct((NC, N, G), jnp.float32),
    mesh=_mesh,
    compiler_params=_sc_params,
    scratch_types=[
        pltpu.VMEM((EPR, CH), jnp.int32),
        pltpu.VMEM((EPR, CH), jnp.int32),
        pltpu.VMEM((2, KP, CH, G), jnp.float32),
        pltpu.MemorySpace.VMEM_SHARED((N, G), jnp.float32),
        pltpu.SemaphoreType.DMA,
        pltpu.SemaphoreType.DMA,
    ],
)
def _propr(src_hbm, dst_hbm, v_hbm, z16_hbm, out_hbm,
           idx_s, idx_d, rows, acc, sem_g, sem_s):
    c = lax.axis_index("c")
    s = lax.axis_index("s")
    wid = c * NS + s
    row0 = wid * EPR
    pltpu.sync_copy(src_hbm.at[pl.ds(row0, EPR)], idx_s)
    pltpu.sync_copy(dst_hbm.at[pl.ds(row0, EPR)], idx_d)

    @pl.when(c == 0)
    def _():
        pltpu.sync_copy(v_hbm.at[pl.ds(s * RPT, RPT)],
                        acc.at[pl.ds(s * RPT, RPT)])

    @pl.when(c != 0)
    def _():
        pltpu.sync_copy(z16_hbm, acc.at[pl.ds(s * RPT, RPT)])

    @pl.when(s == NS - 1)
    def _():
        @pl.when(c == 0)
        def _():
            pltpu.sync_copy(v_hbm.at[pl.ds(NS * RPT, REM)],
                            acc.at[pl.ds(NS * RPT, REM)])

        @pl.when(c != 0)
        def _():
            pltpu.sync_copy(z16_hbm.at[pl.ds(0, REM)],
                            acc.at[pl.ds(NS * RPT, REM)])

    plsc.subcore_barrier()

    def fire_gather(t, b):
        for j in range(KP):
            pltpu.async_copy(v_hbm.at[idx_d.at[t * KP + j]],
                             rows.at[b, j], sem_g)

    def drain_gather():
        for j in range(KP):
            pltpu.make_async_copy(v_hbm.at[idx_d.at[0]],
                                  rows.at[0, j], sem_g).wait()

    def fire_scatter(t, b):
        for j in range(KP):
            pltpu.async_copy(rows.at[b, j], acc.at[idx_s.at[t * KP + j]],
                             sem_s, add=True)

    def drain_scatter():
        for j in range(KP):
            pltpu.make_async_copy(rows.at[0, j],
                                  acc.at[idx_s.at[0]], sem_s).wait()

    fire_gather(0, 0)

    def step(t, carry):
        b = lax.rem(t, 2)
        nb = 1 - b

        @pl.when(t >= 1)
        def _():
            drain_scatter()

        @pl.when(t < TP - 1)
        def _():
            fire_gather(t + 1, nb)

        drain_gather()
        fire_scatter(t, b)
        return carry

    lax.fori_loop(0, TP, step, 0)
    drain_scatter()
    plsc.subcore_barrier()
    pltpu.sync_copy(acc.at[pl.ds(s * RPT, RPT)],
                    out_hbm.at[c, pl.ds(s * RPT, RPT)])

    @pl.when(s == NS - 1)
    def _():
        pltpu.sync_copy(acc.at[pl.ds(NS * RPT, REM)],
                        out_hbm.at[c, pl.ds(NS * RPT, REM)])


@functools.partial(
    pl.kernel,
    out_type=jax.ShapeDtypeStruct((NC, N, G), jnp.float32),
    mesh=_mesh,
    compiler_params=_sc_params,
    scratch_types=[
        pltpu.VMEM((EPR, CH), jnp.int32),
        pltpu.VMEM((CH, G), jnp.float32),
        pltpu.MemorySpace.VMEM_SHARED((N, G), jnp.float32),
        pltpu.SemaphoreType.DMA,
    ],
)
def _deg(dst_hbm, ones_hbm, zeros_hbm, out_hbm, idx_d, ones_v, acc, sem_s):
    c = lax.axis_index("c")
    s = lax.axis_index("s")
    wid = c * NS + s
    pltpu.sync_copy(ones_hbm, ones_v)
    pltpu.sync_copy(dst_hbm.at[pl.ds(wid * EPR, EPR)], idx_d)
    pltpu.sync_copy(zeros_hbm, acc.at[pl.ds(s * RPT, RPT)])

    @pl.when(s == NS - 1)
    def _():
        pltpu.sync_copy(zeros_hbm.at[pl.ds(0, REM)],
                        acc.at[pl.ds(NS * RPT, REM)])

    plsc.subcore_barrier()

    def drain():
        for j in range(KD):
            pltpu.make_async_copy(ones_v, acc.at[idx_d.at[0]], sem_s).wait()

    def step(t, carry):
        @pl.when(t >= 1)
        def _():
            drain()

        for j in range(KD):
            pltpu.async_copy(ones_v, acc.at[idx_d.at[t * KD + j]], sem_s,
                             add=True)
        return carry

    lax.fori_loop(0, TD, step, 0)
    drain()
    plsc.subcore_barrier()
    pltpu.sync_copy(acc.at[pl.ds(s * RPT, RPT)],
                    out_hbm.at[c, pl.ds(s * RPT, RPT)])

    @pl.when(s == NS - 1)
    def _():
        pltpu.sync_copy(acc.at[pl.ds(NS * RPT, REM)],
                        out_hbm.at[c, pl.ds(NS * RPT, REM)])


def _tc1_body(x_ref, w1_ref, degp_ref, batch_ref, xs1_ref, dinvb_ref, v_ref):
    deg = degp_ref[0, :, 0:1] + degp_ref[1, :, 0:1] + 1.0
    dinv = lax.rsqrt(deg)
    xw = jnp.dot(x_ref[...], w1_ref[...], preferred_element_type=jnp.float32)
    xs1_ref[...] = xw * dinv
    dinvb_ref[...] = jnp.broadcast_to(dinv, (N, DH))
    gids = lax.broadcasted_iota(jnp.int32, (N, G), 1)
    onehot = (batch_ref[...] == gids).astype(jnp.float32)
    v_ref[...] = onehot * dinv


def _tc2_body(acc1_ref, xs1_ref, dinvb_ref, m_ref, batch_ref, scal_ref,
              b1_ref, w2_ref, b2_ref, w3_ref, b3_ref, w4_ref, b4_ref,
              out_ref):
    tot = acc1_ref[0] + acc1_ref[1] + xs1_ref[...]
    h1 = jnp.maximum(dinvb_ref[...] * tot + b1_ref[...], 0.0)
    xs2 = (jnp.dot(h1, w2_ref[...], preferred_element_type=jnp.float32)
           * dinvb_ref[:, :DO])
    m = m_ref[0] + m_ref[1]
    sums = lax.dot_general(m, xs2, (((0,), (0,)), ((), ())),
                           preferred_element_type=jnp.float32)
    gids = lax.broadcasted_iota(jnp.int32, (N, G), 1)
    p = (batch_ref[...] == gids).astype(jnp.float32)
    counts = lax.dot_general(p, jnp.ones((N, 1), jnp.float32),
                             (((0,), (0,)), ((), ())),
                             preferred_element_type=jnp.float32)
    pooled = (sums + counts * b2_ref[...]) / jnp.maximum(counts, 1.0)
    comb = jnp.concatenate([pooled, scal_ref[...]], axis=1)
    z = jnp.maximum(
        jnp.dot(comb, w3_ref[...], preferred_element_type=jnp.float32)
        + b3_ref[...], 0.0)
    o = jnp.dot(z, w4_ref[...], preferred_element_type=jnp.float32) + b4_ref[...]
    out_ref[...] = 1.0 / (1.0 + jnp.exp(-o))


_tc1 = pl.pallas_call(
    _tc1_body,
    out_shape=[jax.ShapeDtypeStruct((N, DH), jnp.float32),
               jax.ShapeDtypeStruct((N, DH), jnp.float32),
               jax.ShapeDtypeStruct((N, G), jnp.float32)],
)

_tc2 = pl.pallas_call(
    _tc2_body,
    out_shape=jax.ShapeDtypeStruct((G, 1), jnp.float32),
)


def kernel(x, edge_index, batch, scalars, W1, b1, W2, b2, W3, b3, W4, b4):
    src2d = edge_index[0].astype(jnp.int32).reshape(E // CH, CH)
    dst2d = edge_index[1].astype(jnp.int32).reshape(E // CH, CH)
    batch2d = batch.astype(jnp.int32).reshape(N, 1)
    ones16 = jnp.ones((CH, G), jnp.float32)
    z16 = jnp.zeros((RPT, G), jnp.float32)
    z64 = jnp.zeros((RPT, DH), jnp.float32)

    degp = _deg(dst2d, ones16, z16)
    xs1, dinvb, v = _tc1(x, W1, degp, batch2d)
    acc1 = _prop64(src2d, dst2d, xs1, z64)
    accr = _propr(src2d, dst2d, v, z16)
    out = _tc2(acc1, xs1, dinvb, accr, batch2d, scalars,
               b1.reshape(1, DH), W2, b2.reshape(1, DO),
               W3, b3.reshape(1, DH), W4, b4.reshape(1, 1))
    return out
